# baseline (speedup 1.0000x reference)
import functools

import jax
import jax.numpy as jnp
from jax import lax
from jax.experimental import pallas as pl
from jax.experimental.pallas import tpu as pltpu
from jax.experimental.pallas import tpu_sc as plsc

EMBED_DIM = 32
SEQ = 50
BATCH = 16384
NUM_CORES = 2
NUM_SUBCORES = 16
NUM_WORKERS = NUM_CORES * NUM_SUBCORES
BLK = 128
NUM_UNITS = SEQ * (BATCH // BLK)
UNITS_PER_W = NUM_UNITS // NUM_WORKERS
NBUF = 4
TPITCH = 129


@jax.jit
def _embed_gather(idx_t_flat, weight):
    mesh = plsc.VectorSubcoreMesh(core_axis_name="c", subcore_axis_name="s")

    @functools.partial(
        pl.kernel,
        mesh=mesh,
        out_type=jax.ShapeDtypeStruct((SEQ, 4, BATCH // BLK, 8, BLK), jnp.float32),
        scratch_types=[
            pltpu.VMEM((UNITS_PER_W * BLK,), jnp.int32),
            pltpu.VMEM((NBUF, BLK, EMBED_DIM), jnp.float32),
            pltpu.VMEM((NBUF, 4, 8, TPITCH), jnp.float32),
        ] + [pltpu.SemaphoreType.DMA] * (2 * NBUF),
        compiler_params=pltpu.CompilerParams(
            use_tc_tiling_on_sc=False, needs_layout_passes=False),
    )
    def run(idx_hbm, w_hbm, out_hbm, idx_v, rows, tiles, *sems):
        gsems, ssems = sems[:NBUF], sems[NBUF:]
        wid = lax.axis_index("s") * NUM_CORES + lax.axis_index("c")
        u0 = wid * UNITS_PER_W
        iota = lax.iota(jnp.int32, 16)
        zeros16 = jnp.zeros((16,), jnp.int32)
        tr_vecs = [(c0 + iota) // 8 for c0 in (0, 16)]
        sl_vecs = [(c0 + iota) % 8 for c0 in (0, 16)]

        def g_desc(uu, b):
            return pltpu.make_async_copy(
                w_hbm.at[idx_v.at[pl.ds(uu * BLK, BLK)]], rows.at[b], gsems[b])

        def s_descs(uu, b):
            u = u0 + uu
            s, tc = u // (BATCH // BLK), u % (BATCH // BLK)
            return [
                pltpu.make_async_copy(
                    tiles.at[b, tr, :, pl.ds(0, BLK)],
                    out_hbm.at[s, tr, tc], ssems[b])
                for tr in range(4)
            ]

        pltpu.sync_copy(idx_hbm.at[pl.ds(u0 * BLK, UNITS_PER_W * BLK)], idx_v)
        for b in range(NBUF):
            g_desc(b, b).start()

        def body(t, carry):
            for b in range(NBUF):
                uu = t * NBUF + b

                @pl.when(t > 0)
                def _drain_store(uu=uu, b=b):
                    for d in s_descs(uu, b):
                        d.wait()

                g_desc(uu, b).wait()
                for r in range(BLK):
                    lnr = zeros16 + r
                    for h in range(2):
                        v = rows.at[b][r, pl.ds(h * 16, 16)]
                        plsc.store_scatter(
                            tiles.at[b],
                            [tr_vecs[h], sl_vecs[h], lnr], v)
                for d in s_descs(uu, b):
                    d.start()

                @pl.when(uu + NBUF < UNITS_PER_W)
                def _fire_next(uu=uu, b=b):
                    g_desc(uu + NBUF, b).start()

            return carry

        lax.fori_loop(0, UNITS_PER_W // NBUF, body, 0)
        for b in range(NBUF):
            for d in s_descs(UNITS_PER_W - NBUF + b, b):
                d.wait()

    return run(idx_t_flat, weight)


def kernel(idx, weight):
    idx_t_flat = idx.T.reshape(-1).astype(jnp.int32)
    out5 = _embed_gather(idx_t_flat, weight)
    return out5.transpose(2, 4, 0, 1, 3).reshape(BATCH, SEQ, EMBED_DIM)


# speedup vs baseline: 1.0698x; 1.0698x over previous
import functools

import jax
import jax.numpy as jnp
from jax import lax
from jax.experimental import pallas as pl
from jax.experimental.pallas import tpu as pltpu
from jax.experimental.pallas import tpu_sc as plsc

EMBED_DIM = 32
SEQ = 50
BATCH = 16384
NUM_CORES = 2
NUM_SUBCORES = 16
NUM_WORKERS = NUM_CORES * NUM_SUBCORES
BLK = 128
NUM_UNITS = SEQ * (BATCH // BLK)
UNITS_PER_W = NUM_UNITS // NUM_WORKERS
NBUF = 2
TPITCH = 129


@jax.jit
def _embed_gather(idx_t_flat, weight):
    mesh = plsc.VectorSubcoreMesh(core_axis_name="c", subcore_axis_name="s")

    @functools.partial(
        pl.kernel,
        mesh=mesh,
        out_type=jax.ShapeDtypeStruct((SEQ, 4, BATCH // BLK, 8, BLK), jnp.float32),
        scratch_types=[
            pltpu.VMEM((UNITS_PER_W * BLK,), jnp.int32),
            pltpu.VMEM((NBUF, BLK, EMBED_DIM), jnp.float32),
            pltpu.VMEM((NBUF, 4, 8, TPITCH), jnp.float32),
        ] + [pltpu.SemaphoreType.DMA] * (2 * NBUF),
        compiler_params=pltpu.CompilerParams(
            use_tc_tiling_on_sc=False, needs_layout_passes=False),
    )
    def run(idx_hbm, w_hbm, out_hbm, idx_v, rows, tiles, *sems):
        gsems, ssems = sems[:NBUF], sems[NBUF:]
        wid = lax.axis_index("s") * NUM_CORES + lax.axis_index("c")
        u0 = wid * UNITS_PER_W
        iota = lax.iota(jnp.int32, 16)
        zeros16 = jnp.zeros((16,), jnp.int32)
        tr_vecs = [(c0 + iota) // 8 for c0 in (0, 16)]
        sl_vecs = [(c0 + iota) % 8 for c0 in (0, 16)]

        def g_desc(uu, b):
            return pltpu.make_async_copy(
                w_hbm.at[idx_v.at[pl.ds(uu * BLK, BLK)]], rows.at[b], gsems[b])

        def s_descs(uu, b):
            u = u0 + uu
            s, tc = u // (BATCH // BLK), u % (BATCH // BLK)
            return [
                pltpu.make_async_copy(
                    tiles.at[b, tr, :, pl.ds(0, BLK)],
                    out_hbm.at[s, tr, tc], ssems[b])
                for tr in range(4)
            ]

        pltpu.sync_copy(idx_hbm.at[pl.ds(u0 * BLK, UNITS_PER_W * BLK)], idx_v)
        for b in range(NBUF):
            g_desc(b, b).start()

        def body(t, carry):
            for b in range(NBUF):
                uu = t * NBUF + b

                @pl.when(t > 0)
                def _drain_store(uu=uu, b=b):
                    for d in s_descs(uu, b):
                        d.wait()

                g_desc(uu, b).wait()
                for r in range(BLK):
                    lnr = zeros16 + r
                    for h in range(2):
                        v = rows.at[b][r, pl.ds(h * 16, 16)]
                        plsc.store_scatter(
                            tiles.at[b],
                            [tr_vecs[h], sl_vecs[h], lnr], v)
                for d in s_descs(uu, b):
                    d.start()

                @pl.when(uu + NBUF < UNITS_PER_W)
                def _fire_next(uu=uu, b=b):
                    g_desc(uu + NBUF, b).start()

            return carry

        lax.fori_loop(0, UNITS_PER_W // NBUF, body, 0)
        for b in range(NBUF):
            for d in s_descs(UNITS_PER_W - NBUF + b, b):
                d.wait()

    return run(idx_t_flat, weight)


def kernel(idx, weight):
    idx_t_flat = idx.T.reshape(-1).astype(jnp.int32)
    out5 = _embed_gather(idx_t_flat, weight)
    return out5.transpose(2, 4, 0, 1, 3).reshape(BATCH, SEQ, EMBED_DIM)
